# Initial kernel scaffold; baseline (speedup 1.0000x reference)
import jax
import jax.numpy as jnp
from jax.experimental import pallas as pl


def kernel(x, adj, W_b, b_b, W1, b1, W2, b2, K, alpha):
    raise NotImplementedError("write your pallas kernel here")


# speedup vs baseline: 6.0744x; 6.0744x over previous
import functools

import jax
import jax.numpy as jnp
from jax import lax
from jax.experimental import pallas as pl
from jax.experimental.pallas import tpu as pltpu


def _enc_body(x_ref, wb_ref, bb_ref, w1_ref, s1_ref):
    xm = jnp.mean(x_ref[...], axis=1, keepdims=True)
    h = xm * wb_ref[...] + bb_ref[...]
    s1_ref[...] = jnp.dot(h, w1_ref[...],
                          preferred_element_type=jnp.float32)


def _topk_body(alpha_ref, kcap_ref, adj_ref, s1_ref, b1_ref,
               w_ref, h1_ref):
    a = adj_ref[...]
    bn = a.shape
    colid = lax.broadcasted_iota(jnp.int32, bn, 1)
    alpha = alpha_ref[0]
    kcap = kcap_ref[0]
    work = a
    w = jnp.zeros(bn, jnp.float32)
    for k in range(32):
        m = jnp.min(work, axis=1, keepdims=True)
        cand = jnp.where(work == m, colid, jnp.int32(bn[1]))
        fc = jnp.min(cand, axis=1, keepdims=True)
        sel = colid == fc
        val = jnp.exp(-alpha * m)
        take = jnp.logical_and(sel, k < kcap)
        w = jnp.where(take, val, w)
        work = jnp.where(sel, jnp.float32(jnp.inf), work)
    z = jnp.sum(w, axis=1, keepdims=True)
    wn = w / z
    w_ref[...] = wn
    a1 = jnp.dot(wn, s1_ref[...],
                 preferred_element_type=jnp.float32) + b1_ref[...]
    h1_ref[...] = jnp.maximum(a1, 0.0)


def _l2_body(w_ref, h1_ref, w2_ref, b2_ref, out_ref):
    s2 = jnp.dot(h1_ref[...], w2_ref[...],
                 preferred_element_type=jnp.float32)
    a2 = jnp.dot(w_ref[...], s2,
                 preferred_element_type=jnp.float32) + b2_ref[...]
    mx = jnp.max(a2, axis=1, keepdims=True)
    e = jnp.exp(a2 - mx)
    lse = jnp.log(jnp.sum(e, axis=1, keepdims=True)) + mx
    out_ref[...] = a2 - lse


def kernel(x, adj, W_b, b_b, W1, b1, W2, b2, K, alpha):
    n, c_in, t = x.shape
    d = W1.shape[0]
    nc = W2.shape[1]
    x2 = x.reshape(n, c_in * t)
    bb = b_b.reshape(1, d)
    b1r = b1.reshape(1, d)
    b2r = b2.reshape(1, nc)
    alpha_f = jnp.asarray(alpha, jnp.float32).reshape(1)
    kcap = jnp.asarray(K, jnp.int32).reshape(1)

    benc = 512
    s1 = pl.pallas_call(
        _enc_body,
        grid=(n // benc,),
        in_specs=[
            pl.BlockSpec((benc, c_in * t), lambda i: (i, 0)),
            pl.BlockSpec((c_in, d), lambda i: (0, 0)),
            pl.BlockSpec((1, d), lambda i: (0, 0)),
            pl.BlockSpec((d, d), lambda i: (0, 0)),
        ],
        out_specs=pl.BlockSpec((benc, d), lambda i: (i, 0)),
        out_shape=jax.ShapeDtypeStruct((n, d), jnp.float32),
    )(x2, W_b, bb, W1)

    btop = 128
    grid_spec = pltpu.PrefetchScalarGridSpec(
        num_scalar_prefetch=2,
        grid=(n // btop,),
        in_specs=[
            pl.BlockSpec((btop, n), lambda i, *_: (i, 0)),
            pl.BlockSpec((n, d), lambda i, *_: (0, 0)),
            pl.BlockSpec((1, d), lambda i, *_: (0, 0)),
        ],
        out_specs=[
            pl.BlockSpec((btop, n), lambda i, *_: (i, 0)),
            pl.BlockSpec((btop, d), lambda i, *_: (i, 0)),
        ],
    )
    wdense, h1 = pl.pallas_call(
        _topk_body,
        grid_spec=grid_spec,
        out_shape=[
            jax.ShapeDtypeStruct((n, n), jnp.float32),
            jax.ShapeDtypeStruct((n, d), jnp.float32),
        ],
    )(alpha_f, kcap, adj, s1, b1r)

    bl2 = 512
    out = pl.pallas_call(
        _l2_body,
        grid=(n // bl2,),
        in_specs=[
            pl.BlockSpec((bl2, n), lambda i: (i, 0)),
            pl.BlockSpec((n, d), lambda i: (0, 0)),
            pl.BlockSpec((d, nc), lambda i: (0, 0)),
            pl.BlockSpec((1, nc), lambda i: (0, 0)),
        ],
        out_specs=pl.BlockSpec((bl2, nc), lambda i: (i, 0)),
        out_shape=jax.ShapeDtypeStruct((n, nc), jnp.float32),
    )(wdense, h1, W2, b2r)
    return out


# speedup vs baseline: 13.2487x; 2.1811x over previous
import functools

import jax
import jax.numpy as jnp
from jax import lax
from jax.experimental import pallas as pl
from jax.experimental.pallas import tpu as pltpu


def _enc_body(x_ref, wb_ref, bb_ref, w1_ref, s1_ref):
    xm = jnp.mean(x_ref[...], axis=1, keepdims=True)
    h = xm * wb_ref[...] + bb_ref[...]
    s1_ref[...] = jnp.dot(h, w1_ref[...],
                          preferred_element_type=jnp.float32)


def _topk_body(alpha_ref, kcap_ref, adj_ref, s1_ref, b1_ref,
               w_ref, h1_ref):
    a = adj_ref[...]
    bn = a.shape
    colid = lax.broadcasted_iota(jnp.int32, bn, 1)
    alpha = alpha_ref[0]
    kk = jnp.minimum(kcap_ref[0], 32)
    key = lax.bitcast_convert_type(a, jnp.int32)
    p = jnp.zeros((bn[0], 1), jnp.int32)
    for b in range(29, -1, -1):
        t = p + jnp.int32(1 << b)
        c = jnp.sum((key < t).astype(jnp.int32), axis=1, keepdims=True)
        p = jnp.where(c < kk, t, p)
    less = key < p
    eq = key == p
    c_less = jnp.sum(less.astype(jnp.int32), axis=1, keepdims=True)
    m = kk - c_less
    q = jnp.zeros((bn[0], 1), jnp.int32)
    for b in range(11, -1, -1):
        t = q + jnp.int32(1 << b)
        c = jnp.sum(jnp.where(eq & (colid < t), 1, 0),
                    axis=1, keepdims=True)
        q = jnp.where(c < m, t, q)
    sel = less | (eq & (colid <= q))
    w = jnp.where(sel, jnp.exp(-alpha * a), 0.0)
    z = jnp.sum(w, axis=1, keepdims=True)
    wn = w / z
    w_ref[...] = wn
    a1 = jnp.dot(wn, s1_ref[...],
                 preferred_element_type=jnp.float32) + b1_ref[...]
    h1_ref[...] = jnp.maximum(a1, 0.0)


def _l2_body(w_ref, h1_ref, w2_ref, b2_ref, out_ref):
    s2 = jnp.dot(h1_ref[...], w2_ref[...],
                 preferred_element_type=jnp.float32)
    a2 = jnp.dot(w_ref[...], s2,
                 preferred_element_type=jnp.float32) + b2_ref[...]
    mx = jnp.max(a2, axis=1, keepdims=True)
    e = jnp.exp(a2 - mx)
    lse = jnp.log(jnp.sum(e, axis=1, keepdims=True)) + mx
    out_ref[...] = a2 - lse


def kernel(x, adj, W_b, b_b, W1, b1, W2, b2, K, alpha):
    n, c_in, t = x.shape
    d = W1.shape[0]
    nc = W2.shape[1]
    x2 = x.reshape(n, c_in * t)
    bb = b_b.reshape(1, d)
    b1r = b1.reshape(1, d)
    b2r = b2.reshape(1, nc)
    alpha_f = jnp.asarray(alpha, jnp.float32).reshape(1)
    kcap = jnp.asarray(K, jnp.int32).reshape(1)

    benc = 512
    s1 = pl.pallas_call(
        _enc_body,
        grid=(n // benc,),
        in_specs=[
            pl.BlockSpec((benc, c_in * t), lambda i: (i, 0)),
            pl.BlockSpec((c_in, d), lambda i: (0, 0)),
            pl.BlockSpec((1, d), lambda i: (0, 0)),
            pl.BlockSpec((d, d), lambda i: (0, 0)),
        ],
        out_specs=pl.BlockSpec((benc, d), lambda i: (i, 0)),
        out_shape=jax.ShapeDtypeStruct((n, d), jnp.float32),
    )(x2, W_b, bb, W1)

    btop = 128
    grid_spec = pltpu.PrefetchScalarGridSpec(
        num_scalar_prefetch=2,
        grid=(n // btop,),
        in_specs=[
            pl.BlockSpec((btop, n), lambda i, *_: (i, 0)),
            pl.BlockSpec((n, d), lambda i, *_: (0, 0)),
            pl.BlockSpec((1, d), lambda i, *_: (0, 0)),
        ],
        out_specs=[
            pl.BlockSpec((btop, n), lambda i, *_: (i, 0)),
            pl.BlockSpec((btop, d), lambda i, *_: (i, 0)),
        ],
    )
    wdense, h1 = pl.pallas_call(
        _topk_body,
        grid_spec=grid_spec,
        out_shape=[
            jax.ShapeDtypeStruct((n, n), jnp.float32),
            jax.ShapeDtypeStruct((n, d), jnp.float32),
        ],
    )(alpha_f, kcap, adj, s1, b1r)

    bl2 = 512
    out = pl.pallas_call(
        _l2_body,
        grid=(n // bl2,),
        in_specs=[
            pl.BlockSpec((bl2, n), lambda i: (i, 0)),
            pl.BlockSpec((n, d), lambda i: (0, 0)),
            pl.BlockSpec((d, nc), lambda i: (0, 0)),
            pl.BlockSpec((1, nc), lambda i: (0, 0)),
        ],
        out_specs=pl.BlockSpec((bl2, nc), lambda i: (i, 0)),
        out_shape=jax.ShapeDtypeStruct((n, nc), jnp.float32),
    )(wdense, h1, W2, b2r)
    return out
